# baseline (speedup 1.0000x reference)
import jax
import jax.numpy as jnp
from jax import lax
from jax.experimental import pallas as pl
from jax.experimental.pallas import tpu as pltpu

F32 = jnp.float32
_BE = 2000
_BN = 2000


def _ln(x, s, b):
    mu = jnp.mean(x, axis=-1, keepdims=True)
    xc = x - mu
    var = jnp.mean(xc * xc, axis=-1, keepdims=True)
    return xc * lax.rsqrt(var + 1e-5) * s + b


def _mm(x, w):
    return jnp.dot(x, w, preferred_element_type=F32,
                   precision=lax.Precision.HIGHEST)


def _full(shape):
    return pl.BlockSpec(shape, lambda i: (0,) * len(shape))


def _rows(bs, w):
    return pl.BlockSpec((bs, w), lambda i: (i, 0))


def _node_enc(vel, ntype, w1v, wtype, w2, b2, w3, b3, lns, lnb):
    n = vel.shape[0]

    def body(vel_ref, t_ref, w1v_ref, wt_ref, w2_ref, b2_ref, w3_ref, b3_ref,
             s_ref, b_ref, o_ref):
        v = vel_ref[...]
        t = v[:, 0:1] * w1v_ref[0:1, :] + v[:, 1:2] * w1v_ref[1:2, :]
        tt = t_ref[...]
        for k in range(9):
            t = t + jnp.where(tt == k, 1.0, 0.0) * wt_ref[k:k + 1, :]
        t = jnp.maximum(t, 0.0)
        t = jnp.maximum(_mm(t, w2_ref[...]) + b2_ref[...], 0.0)
        t = _mm(t, w3_ref[...]) + b3_ref[...]
        o_ref[...] = _ln(t, s_ref[...], b_ref[...])

    return pl.pallas_call(
        body,
        grid=(n // _BN,),
        in_specs=[_rows(_BN, 2), _rows(_BN, 1), _full((2, 128)), _full((9, 128)),
                  _full((128, 128)), _full((1, 128)), _full((128, 128)),
                  _full((1, 128)), _full((1, 128)), _full((1, 128))],
        out_specs=_rows(_BN, 128),
        out_shape=jax.ShapeDtypeStruct((n, 128), F32),
    )(vel, ntype, w1v, wtype, w2, b2, w3, b3, lns, lnb)


def _edge_enc(sp, dp, w1, b1, w2, b2, w3, b3, lns, lnb):
    e = sp.shape[0]

    def body(sp_ref, dp_ref, w1_ref, b1_ref, w2_ref, b2_ref, w3_ref, b3_ref,
             s_ref, bb_ref, o_ref):
        r = sp_ref[...] - dp_ref[...]
        rx = r[:, 0:1]
        ry = r[:, 1:2]
        rn = jnp.sqrt(rx * rx + ry * ry)
        t = rx * w1_ref[0:1, :] + ry * w1_ref[1:2, :] + rn * w1_ref[2:3, :] + b1_ref[...]
        t = jnp.maximum(t, 0.0)
        t = jnp.maximum(_mm(t, w2_ref[...]) + b2_ref[...], 0.0)
        t = _mm(t, w3_ref[...]) + b3_ref[...]
        o_ref[...] = _ln(t, s_ref[...], bb_ref[...])

    return pl.pallas_call(
        body,
        grid=(e // _BE,),
        in_specs=[_rows(_BE, 2), _rows(_BE, 2), _full((3, 128)), _full((1, 128)),
                  _full((128, 128)), _full((1, 128)), _full((128, 128)),
                  _full((1, 128)), _full((1, 128)), _full((1, 128))],
        out_specs=_rows(_BE, 128),
        out_shape=jax.ShapeDtypeStruct((e, 128), F32),
    )(sp, dp, w1, b1, w2, b2, w3, b3, lns, lnb)


def _proj2(x, ws, wd):
    n = x.shape[0]

    def body(x_ref, ws_ref, wd_ref, os_ref, od_ref):
        xv = x_ref[...]
        os_ref[...] = _mm(xv, ws_ref[...])
        od_ref[...] = _mm(xv, wd_ref[...])

    return pl.pallas_call(
        body,
        grid=(n // _BN,),
        in_specs=[_rows(_BN, 128), _full((128, 128)), _full((128, 128))],
        out_specs=[_rows(_BN, 128), _rows(_BN, 128)],
        out_shape=[jax.ShapeDtypeStruct((n, 128), F32)] * 2,
    )(x, ws, wd)


def _edge_step(he, gs, gd, w1e, b1, w2, b2, w3, b3, lns, lnb):
    e = he.shape[0]

    def body(he_ref, gs_ref, gd_ref, w1e_ref, b1_ref, w2_ref, b2_ref, w3_ref,
             b3_ref, s_ref, bb_ref, o_ref):
        he_v = he_ref[...]
        t = _mm(he_v, w1e_ref[...]) + gs_ref[...] + gd_ref[...] + b1_ref[...]
        t = jnp.maximum(t, 0.0)
        t = jnp.maximum(_mm(t, w2_ref[...]) + b2_ref[...], 0.0)
        t = _mm(t, w3_ref[...]) + b3_ref[...]
        o_ref[...] = he_v + _ln(t, s_ref[...], bb_ref[...])

    return pl.pallas_call(
        body,
        grid=(e // _BE,),
        in_specs=[_rows(_BE, 128), _rows(_BE, 128), _rows(_BE, 128),
                  _full((128, 128)), _full((1, 128)), _full((128, 128)),
                  _full((1, 128)), _full((128, 128)), _full((1, 128)),
                  _full((1, 128)), _full((1, 128))],
        out_specs=_rows(_BE, 128),
        out_shape=jax.ShapeDtypeStruct((e, 128), F32),
    )(he, gs, gd, w1e, b1, w2, b2, w3, b3, lns, lnb)


def _node_step(hn, agg, w1a, w1b, b1, w2, b2, w3, b3, lns, lnb):
    n = hn.shape[0]

    def body(hn_ref, agg_ref, w1a_ref, w1b_ref, b1_ref, w2_ref, b2_ref, w3_ref,
             b3_ref, s_ref, bb_ref, o_ref):
        hn_v = hn_ref[...]
        t = _mm(hn_v, w1a_ref[...]) + _mm(agg_ref[...], w1b_ref[...]) + b1_ref[...]
        t = jnp.maximum(t, 0.0)
        t = jnp.maximum(_mm(t, w2_ref[...]) + b2_ref[...], 0.0)
        t = _mm(t, w3_ref[...]) + b3_ref[...]
        o_ref[...] = hn_v + _ln(t, s_ref[...], bb_ref[...])

    return pl.pallas_call(
        body,
        grid=(n // _BN,),
        in_specs=[_rows(_BN, 128), _rows(_BN, 128),
                  _full((128, 128)), _full((128, 128)), _full((1, 128)),
                  _full((128, 128)), _full((1, 128)), _full((128, 128)),
                  _full((1, 128)), _full((1, 128)), _full((1, 128))],
        out_specs=_rows(_BN, 128),
        out_shape=jax.ShapeDtypeStruct((n, 128), F32),
    )(hn, agg, w1a, w1b, b1, w2, b2, w3, b3, lns, lnb)


def _decoder3(hn, w1, b1, w2, b2, w3, b3):
    n = hn.shape[0]

    def body(x_ref, w1_ref, b1_ref, w2_ref, b2_ref, w3_ref, b3_ref, o_ref):
        t = jnp.maximum(_mm(x_ref[...], w1_ref[...]) + b1_ref[...], 0.0)
        t = jnp.maximum(_mm(t, w2_ref[...]) + b2_ref[...], 0.0)
        o_ref[...] = _mm(t, w3_ref[...]) + b3_ref[...]

    return pl.pallas_call(
        body,
        grid=(n // _BN,),
        in_specs=[_rows(_BN, 128), _full((128, 128)), _full((1, 128)),
                  _full((128, 128)), _full((1, 128)), _full((128, 2)),
                  _full((1, 2))],
        out_specs=_rows(_BN, 2),
        out_shape=jax.ShapeDtypeStruct((n, 2), F32),
    )(hn, w1, b1, w2, b2, w3, b3)


def kernel(velocity, node_type, cells, mesh_pos, params):
    p = params
    n = velocity.shape[0]
    c0, c1, c2 = cells[:, 0], cells[:, 1], cells[:, 2]
    srcs = jnp.concatenate([c0, c1, c2, c1, c2, c0])
    dsts = jnp.concatenate([c1, c2, c0, c0, c1, c2])
    order = jnp.argsort(dsts)
    srcs = srcs[order].astype(jnp.int32)
    dsts = dsts[order].astype(jnp.int32)

    def r2(b):
        return b.reshape(1, -1)

    nmean, nstd = p['node_norm_mean'], p['node_norm_std']
    (w1n, b1n), (w2n, b2n), (w3n, b3n) = p['node_enc']
    w1n_f = w1n / nstd[:, None]
    b1n_f = b1n - (nmean / nstd) @ w1n
    w1v = w1n_f[:2]
    wtype = w1n_f[2:] + b1n_f[None, :]
    lns_n, lnb_n = p['node_enc_ln']
    h_n = _node_enc(velocity, node_type.reshape(-1, 1).astype(jnp.int32),
                    w1v, wtype, w2n, r2(b2n), w3n, r2(b3n), r2(lns_n), r2(lnb_n))

    emean, estd = p['edge_norm_mean'], p['edge_norm_std']
    (w1e, b1e), (w2e, b2e), (w3e, b3e) = p['edge_enc']
    w1e_f = w1e / estd[:, None]
    b1e_f = b1e - (emean / estd) @ w1e
    lns_e, lnb_e = p['edge_enc_ln']
    sp = jnp.take(mesh_pos, srcs, axis=0)
    dp = jnp.take(mesh_pos, dsts, axis=0)
    h_e = _edge_enc(sp, dp, w1e_f, r2(b1e_f), w2e, r2(b2e), w3e, r2(b3e),
                    r2(lns_e), r2(lnb_e))

    def stack(getter):
        return jnp.stack([getter(i) for i in range(len(p['mp_edge']))])

    xs = {
        'ew1e': stack(lambda i: p['mp_edge'][i][0][0][:128]),
        'ew1s': stack(lambda i: p['mp_edge'][i][0][0][128:256]),
        'ew1d': stack(lambda i: p['mp_edge'][i][0][0][256:]),
        'eb1': stack(lambda i: r2(p['mp_edge'][i][0][1])),
        'ew2': stack(lambda i: p['mp_edge'][i][1][0]),
        'eb2': stack(lambda i: r2(p['mp_edge'][i][1][1])),
        'ew3': stack(lambda i: p['mp_edge'][i][2][0]),
        'eb3': stack(lambda i: r2(p['mp_edge'][i][2][1])),
        'elns': stack(lambda i: r2(p['mp_edge_ln'][i][0])),
        'elnb': stack(lambda i: r2(p['mp_edge_ln'][i][1])),
        'nw1a': stack(lambda i: p['mp_node'][i][0][0][:128]),
        'nw1b': stack(lambda i: p['mp_node'][i][0][0][128:]),
        'nb1': stack(lambda i: r2(p['mp_node'][i][0][1])),
        'nw2': stack(lambda i: p['mp_node'][i][1][0]),
        'nb2': stack(lambda i: r2(p['mp_node'][i][1][1])),
        'nw3': stack(lambda i: p['mp_node'][i][2][0]),
        'nb3': stack(lambda i: r2(p['mp_node'][i][2][1])),
        'nlns': stack(lambda i: r2(p['mp_node_ln'][i][0])),
        'nlnb': stack(lambda i: r2(p['mp_node_ln'][i][1])),
    }

    def step(carry, w):
        h_n, h_e = carry
        gsf, gdf = _proj2(h_n, w['ew1s'], w['ew1d'])
        gs = jnp.take(gsf, srcs, axis=0)
        gd = jnp.take(gdf, dsts, axis=0)
        h_e = _edge_step(h_e, gs, gd, w['ew1e'], w['eb1'], w['ew2'], w['eb2'],
                         w['ew3'], w['eb3'], w['elns'], w['elnb'])
        agg = jax.ops.segment_sum(h_e, dsts, num_segments=n)
        h_n = _node_step(h_n, agg, w['nw1a'], w['nw1b'], w['nb1'], w['nw2'],
                         w['nb2'], w['nw3'], w['nb3'], w['nlns'], w['nlnb'])
        return (h_n, h_e), None

    (h_n, h_e), _ = lax.scan(step, (h_n, h_e), xs)

    (w1d, b1d), (w2d, b2d), (w3d, b3d) = p['decoder']
    w3d_f = w3d * p['out_norm_std'][None, :]
    b3d_f = b3d * p['out_norm_std'] + p['out_norm_mean']
    return _decoder3(h_n, w1d, r2(b1d), w2d, r2(b2d), w3d_f, r2(b3d_f))
